# Initial kernel scaffold; baseline (speedup 1.0000x reference)
import jax
import jax.numpy as jnp
from jax.experimental import pallas as pl


def kernel(inputs, w):
    raise NotImplementedError("write your pallas kernel here")


# speedup vs baseline: 148.7309x; 148.7309x over previous
import functools

import jax
import jax.numpy as jnp
from jax import lax
from jax.experimental import pallas as pl
from jax.experimental.pallas import tpu as pltpu
from jax.experimental.pallas import tpu_sc as plsc

_B = 16384
_U = 64
_D = 128
_NC = 2
_NS = 16
_NW = _NC * _NS
_BPW = _B // _NW
_L = 16
_UB = _U // _L
_CH = 256


@functools.partial(
    pl.kernel,
    out_type=jax.ShapeDtypeStruct((_B, _U), jnp.float32),
    mesh=plsc.VectorSubcoreMesh(core_axis_name="c", subcore_axis_name="s"),
    scratch_types=[
        pltpu.VMEM((_CH, _D), jnp.float32),
        pltpu.VMEM((_D, _U), jnp.float32),
        pltpu.VMEM((_CH, _U), jnp.float32),
    ],
)
def _trop_top2_sc(x_hbm, wt_hbm, out_hbm, x_v, wt_v, o_v):
    wid = lax.axis_index("s") * _NC + lax.axis_index("c")
    base = wid * _BPW
    pltpu.sync_copy(wt_hbm, wt_v)

    neg = jnp.full((_L,), -jnp.inf, jnp.float32)

    def row(b, carry):
        def jstep(jc, ms):
            m1s = list(ms[:_UB])
            m2s = list(ms[_UB:])
            xv = x_v[b, pl.ds(jc * _L, _L)]
            for jj in range(_L):
                xs = xv[jj]
                j = jc * _L + jj
                for ub in range(_UB):
                    v = wt_v[j, pl.ds(ub * _L, _L)] + xs
                    m2s[ub] = jnp.maximum(m2s[ub], jnp.minimum(m1s[ub], v))
                    m1s[ub] = jnp.maximum(m1s[ub], v)
            return tuple(m1s) + tuple(m2s)

        ms = lax.fori_loop(
            0, _D // _L, jstep, tuple(neg for _ in range(2 * _UB))
        )
        for ub in range(_UB):
            o_v[b, pl.ds(ub * _L, _L)] = ms[ub] - ms[_UB + ub]
        return carry

    for chunk in range(_BPW // _CH):
        cbase = base + chunk * _CH
        pltpu.sync_copy(x_hbm.at[pl.ds(cbase, _CH)], x_v)
        lax.fori_loop(0, _CH, row, 0)
        pltpu.sync_copy(o_v, out_hbm.at[pl.ds(cbase, _CH)])


def kernel(inputs, w):
    return _trop_top2_sc(inputs, w.T)


# speedup vs baseline: 155.1943x; 1.0435x over previous
import functools

import jax
import jax.numpy as jnp
from jax import lax
from jax.experimental import pallas as pl
from jax.experimental.pallas import tpu as pltpu
from jax.experimental.pallas import tpu_sc as plsc

_B = 16384
_U = 64
_D = 128
_NC = 2
_NS = 16
_NW = _NC * _NS
_BPW = _B // _NW
_L = 16
_UB = _U // _L
_CH = 256
_NCK = _D // _L


def _scalar(v):
    return v[0] if getattr(v, "shape", ()) == (_L,) else v


_GATHER_DNUMS = lax.GatherDimensionNumbers(
    offset_dims=(), collapsed_slice_dims=(0,), start_index_map=(0,)
)


def _shuffle(v, idx):
    return lax.gather(
        v, idx[:, None], _GATHER_DNUMS, (1,),
        mode=lax.GatherScatterMode.PROMISE_IN_BOUNDS,
    )


def _max_all(v, lanes):
    for s in (1, 2, 4, 8):
        v = jnp.maximum(v, _shuffle(v, lanes ^ s))
    return v[0]


def _min_all(v, lanes):
    for s in (1, 2, 4, 8):
        v = jnp.minimum(v, _shuffle(v, lanes ^ s))
    return v[0]


def _sum_all(v, lanes):
    for s in (1, 2, 4, 8):
        v = v + _shuffle(v, lanes ^ s)
    return v[0]


@functools.partial(
    pl.kernel,
    out_type=jax.ShapeDtypeStruct((_B, _U), jnp.float32),
    mesh=plsc.VectorSubcoreMesh(core_axis_name="c", subcore_axis_name="s"),
    scratch_types=[
        pltpu.VMEM((_CH, _D), jnp.float32),
        pltpu.VMEM((_D, _U), jnp.float32),
        pltpu.VMEM((_CH, _U), jnp.float32),
    ],
)
def _trop_top2_sc(x_hbm, wt_hbm, out_hbm, x_v, wt_v, o_v):
    wid = lax.axis_index("s") * _NC + lax.axis_index("c")
    base = wid * _BPW
    pltpu.sync_copy(wt_hbm, wt_v)

    neg = jnp.full((_L,), -jnp.inf, jnp.float32)
    pos = jnp.full((_L,), jnp.inf, jnp.float32)
    lanes = lax.broadcasted_iota(jnp.int32, (_L,), 0)

    def wscan(j, carry):
        wmx, wmn = carry
        for ub in range(_UB):
            vv = wt_v[j, pl.ds(ub * _L, _L)]
            wmx = jnp.maximum(wmx, vv)
            wmn = jnp.minimum(wmn, vv)
        return wmx, wmn

    wmx, wmn = lax.fori_loop(0, _D, wscan, (neg, pos))
    w_spread = _max_all(wmx, lanes) + _max_all(-wmn, lanes)

    def row(b, carry):
        xvs = [x_v[b, pl.ds(c * _L, _L)] for c in range(_NCK)]
        h1 = xvs[0]
        h2 = xvs[_NCK // 2]
        for c in range(1, _NCK // 2):
            h1 = jnp.maximum(h1, xvs[c])
            h2 = jnp.maximum(h2, xvs[_NCK // 2 + c])
        x2_lb = jnp.minimum(_max_all(h1, lanes), _max_all(h2, lanes))
        thresh = x2_lb - w_spread

        m1s = [neg] * _UB
        m2s = [neg] * _UB
        for c in range(_NCK):
            mask = xvs[c] >= thresh
            n_c = _sum_all(jnp.where(mask, 1, 0).astype(jnp.int32), lanes)

            def cand(i, cr, c=c):
                xc = cr[0]
                cm1 = list(cr[1:1 + _UB])
                cm2 = list(cr[1 + _UB:])
                l = _min_all(jnp.where(xc >= thresh, lanes, _L), lanes)
                xsv = _shuffle(xc, jnp.full((_L,), l, jnp.int32))
                j = c * _L + l
                for ub in range(_UB):
                    v = wt_v[j, pl.ds(ub * _L, _L)] + xsv
                    lo = jnp.minimum(cm1[ub], v)
                    cm1[ub] = jnp.maximum(cm1[ub], v)
                    cm2[ub] = jnp.maximum(cm2[ub], lo)
                xc = jnp.where(lanes == l, neg, xc)
                return (xc,) + tuple(cm1) + tuple(cm2)

            res = lax.fori_loop(
                0, n_c, cand, (xvs[c],) + tuple(m1s) + tuple(m2s)
            )
            m1s = list(res[1:1 + _UB])
            m2s = list(res[1 + _UB:])

        for ub in range(_UB):
            o_v[b, pl.ds(ub * _L, _L)] = m1s[ub] - m2s[ub]
        return carry

    for chunk in range(_BPW // _CH):
        cbase = base + chunk * _CH
        pltpu.sync_copy(x_hbm.at[pl.ds(cbase, _CH)], x_v)
        lax.fori_loop(0, _CH, row, 0)
        pltpu.sync_copy(o_v, out_hbm.at[pl.ds(cbase, _CH)])


def kernel(inputs, w):
    return _trop_top2_sc(inputs, w.T)


# speedup vs baseline: 311.3506x; 2.0062x over previous
import functools

import jax
import jax.numpy as jnp
from jax import lax
from jax.experimental import pallas as pl
from jax.experimental.pallas import tpu as pltpu
from jax.experimental.pallas import tpu_sc as plsc

_B = 16384
_U = 64
_D = 128
_NC = 2
_NS = 16
_NW = _NC * _NS
_BPW = _B // _NW
_L = 16
_UB = _U // _L
_CH = 256
_NG = _CH // _L
_T = 8
_NBM = _D // 32

_I32 = jnp.int32
_F32 = jnp.float32

_GATHER_DNUMS = lax.GatherDimensionNumbers(
    offset_dims=(), collapsed_slice_dims=(0,), start_index_map=(0,)
)


def _shuffle(v, idx):
    return lax.gather(
        v, idx[:, None], _GATHER_DNUMS, (1,),
        mode=lax.GatherScatterMode.PROMISE_IN_BOUNDS,
    )


def _max_all(v, lanes):
    for s in (1, 2, 4, 8):
        v = jnp.maximum(v, _shuffle(v, lanes ^ s))
    return v[0]


def _popcount32(v):
    shr = lax.shift_right_logical
    v = v - (shr(v, 1) & _I32(0x55555555))
    v = (v & _I32(0x33333333)) + (shr(v, 2) & _I32(0x33333333))
    v = (v + shr(v, 4)) & _I32(0x0F0F0F0F)
    return shr(v * _I32(0x01010101), 24)


def _pop_candidate(bms, rg, r_splat, x_t_v, neg):
    nz = [b != 0 for b in bms]
    sel = jnp.where(
        nz[0], bms[0], jnp.where(nz[1], bms[1], jnp.where(nz[2], bms[2], bms[3]))
    )
    base = jnp.where(
        nz[0],
        _I32(0),
        jnp.where(nz[1], _I32(32), jnp.where(nz[2], _I32(64), _I32(96))),
    )
    low = sel & (-sel)
    bits = lax.bitcast_convert_type(low.astype(_F32), _I32)
    jloc = ((bits >> 23) & 255) - 127
    found = sel != _I32(0)
    j = jnp.where(found, base + jloc, _I32(0))
    cleared = sel & (sel - 1)
    g0 = nz[0]
    g1 = jnp.logical_and(jnp.logical_not(nz[0]), nz[1])
    g2 = jnp.logical_and(jnp.logical_not(jnp.logical_or(nz[0], nz[1])), nz[2])
    g3 = jnp.logical_not(jnp.logical_or(jnp.logical_or(nz[0], nz[1]), nz[2]))
    new_bms = (
        jnp.where(g0, cleared, bms[0]),
        jnp.where(g1, cleared, bms[1]),
        jnp.where(g2, cleared, bms[2]),
        jnp.where(g3, cleared, bms[3]),
    )
    xrow = x_t_v[j, pl.ds(rg * _L, _L)]
    xs_vec = jnp.where(found, _shuffle(xrow, r_splat), neg)
    return new_bms, j, xs_vec


@functools.partial(
    pl.kernel,
    out_type=jax.ShapeDtypeStruct((_B, _U), jnp.float32),
    mesh=plsc.VectorSubcoreMesh(core_axis_name="c", subcore_axis_name="s"),
    scratch_types=[
        pltpu.VMEM((_D, _CH), jnp.float32),
        pltpu.VMEM((_D, _U), jnp.float32),
        pltpu.VMEM((_CH, _U), jnp.float32),
    ],
)
def _trop_top2_sc(xt_hbm, wt_hbm, out_hbm, x_t_v, wt_v, o_v):
    wid = lax.axis_index("s") * _NC + lax.axis_index("c")
    base = wid * _BPW
    pltpu.sync_copy(wt_hbm, wt_v)

    neg = jnp.full((_L,), -jnp.inf, _F32)
    pos = jnp.full((_L,), jnp.inf, _F32)
    zero_i = jnp.zeros((_L,), _I32)
    lanes = lax.broadcasted_iota(_I32, (_L,), 0)

    def wscan(jj, carry):
        wmx, wmn = carry
        for ub in range(_UB):
            vv = wt_v[jj, pl.ds(ub * _L, _L)]
            wmx = jnp.maximum(wmx, vv)
            wmn = jnp.minimum(wmn, vv)
        return wmx, wmn

    wmx, wmn = lax.fori_loop(0, _D, wscan, (neg, pos))
    w_spread = _max_all(wmx, lanes) + _max_all(-wmn, lanes)
    w_spread_vec = jnp.full((_L,), 0.0, _F32) + w_spread

    def rowgroup(rg, carry):
        cols = pl.ds(rg * _L, _L)
        def topx(jj, ms):
            m1, m2 = ms
            v = x_t_v[jj, cols]
            lo = jnp.minimum(m1, v)
            return jnp.maximum(m1, v), jnp.maximum(m2, lo)

        xm1, xm2 = lax.fori_loop(0, _D, topx, (neg, neg))
        thresh = xm2 - w_spread_vec

        bmv = []
        for g in range(_NBM):
            def bmstep(kk, bm, g=g):
                for k8 in range(8):
                    one = jnp.full((_L,), _I32(1))
                    sh = jnp.full((_L,), kk * 8 + k8, _I32)
                    jj = g * 32 + kk * 8 + k8
                    bit = lax.shift_left(one, sh)
                    bm = bm | jnp.where(x_t_v[jj, cols] >= thresh, bit, zero_i)
                return bm

            bmv.append(lax.fori_loop(0, 4, bmstep, zero_i))
        n_vec = zero_i
        for g in range(_NBM):
            n_vec = n_vec + _popcount32(bmv[g])

        def rowbody(r, rcarry):
            rot = (lanes + r) & (_L - 1)
            r_splat = jnp.full((_L,), r, _I32)
            bms = tuple(_shuffle(bmv[g], rot)[0] for g in range(_NBM))
            n_all = _shuffle(n_vec, rot)[0]

            def body(i, c):
                cbms = c[:_NBM]
                cm1 = list(c[_NBM:_NBM + _UB])
                cm2 = list(c[_NBM + _UB:])
                cbms, j2, xs2 = _pop_candidate(cbms, rg, r_splat, x_t_v, neg)
                for ub in range(_UB):
                    v = wt_v[j2, pl.ds(ub * _L, _L)] + xs2
                    lo = jnp.minimum(cm1[ub], v)
                    cm1[ub] = jnp.maximum(cm1[ub], v)
                    cm2[ub] = jnp.maximum(cm2[ub], lo)
                return cbms + tuple(cm1) + tuple(cm2)

            init = bms + tuple([neg] * _UB) + tuple([neg] * _UB)
            res = lax.fori_loop(0, n_all, body, init)
            m1s = res[_NBM:_NBM + _UB]
            m2s = res[_NBM + _UB:]

            for ub in range(_UB):
                o_v[rg * _L + r, pl.ds(ub * _L, _L)] = m1s[ub] - m2s[ub]
            return rcarry

        lax.fori_loop(0, _L, rowbody, 0)
        return carry

    for chunk in range(_BPW // _CH):
        cbase = base + chunk * _CH
        pltpu.sync_copy(xt_hbm.at[:, pl.ds(cbase, _CH)], x_t_v)
        lax.fori_loop(0, _NG, rowgroup, 0)
        pltpu.sync_copy(o_v, out_hbm.at[pl.ds(cbase, _CH)])


def kernel(inputs, w):
    return _trop_top2_sc(inputs.T, w.T)
